# Initial kernel scaffold; baseline (speedup 1.0000x reference)
import jax
import jax.numpy as jnp
from jax.experimental import pallas as pl


def kernel(x, indices):
    raise NotImplementedError("write your pallas kernel here")


# speedup vs baseline: 91.6638x; 91.6638x over previous
import functools

import jax
import jax.numpy as jnp
from jax import lax
from jax.experimental import pallas as pl
from jax.experimental.pallas import tpu as pltpu
from jax.experimental.pallas import tpu_sc as plsc

B, C, H, W = 4, 96, 384, 384
Hp, Wp = H // 2, W // 2
P = B * C
NW = 32
PPW = P // NW
R = 96
NCH = Hp // R
IN_CH = R * Wp
OUT_CH = 2 * R * W

_mesh = plsc.VectorSubcoreMesh(core_axis_name="c", subcore_axis_name="s")


@functools.partial(
    pl.kernel,
    mesh=_mesh,
    out_type=jax.ShapeDtypeStruct((P * H * W,), jnp.float32),
    scratch_types=[
        pltpu.VMEM((IN_CH,), jnp.float32),
        pltpu.VMEM((IN_CH,), jnp.int32),
        pltpu.VMEM((OUT_CH,), jnp.float32),
        pltpu.SemaphoreType.DMA,
    ],
    compiler_params=pltpu.CompilerParams(needs_layout_passes=False),
)
def _unpool(x_hbm, idx_hbm, out_hbm, x_v, idx_v, out_v, sem):
    wid = lax.axis_index("s") * 2 + lax.axis_index("c")

    def plane_body(t, _):
        plane = wid * PPW + t

        def chunk_body(ci, _):
            in_off = plane * (Hp * Wp) + ci * IN_CH
            out_off = plane * (H * W) + ci * OUT_CH
            cp_x = pltpu.async_copy(x_hbm.at[pl.ds(in_off, IN_CH)], x_v, sem)
            cp_i = pltpu.async_copy(idx_hbm.at[pl.ds(in_off, IN_CH)], idx_v, sem)

            zeros = jnp.zeros((16,), jnp.float32)

            def zbody(k, _):
                out_v[pl.ds(k * 16, 16)] = zeros
                return ()

            lax.fori_loop(0, OUT_CH // 16, zbody, (), unroll=8)
            cp_x.wait()
            cp_i.wait()

            base = jnp.full((16,), ci * OUT_CH, jnp.int32)

            def sbody(k, _):
                iv = idx_v[pl.ds(k * 16, 16)] - base
                xv = x_v[pl.ds(k * 16, 16)]
                plsc.store_scatter(out_v, [iv], xv)
                return ()

            lax.fori_loop(0, IN_CH // 16, sbody, (), unroll=8)
            pltpu.sync_copy(out_v, out_hbm.at[pl.ds(out_off, OUT_CH)])
            return ()

        lax.fori_loop(0, NCH, chunk_body, ())
        return ()

    lax.fori_loop(0, PPW, plane_body, ())


def kernel(x, indices):
    out = _unpool(x.reshape(-1), indices.reshape(-1))
    return out.reshape(B, C, H, W)


# speedup vs baseline: 100.5644x; 1.0971x over previous
import functools

import jax
import jax.numpy as jnp
from jax import lax
from jax.experimental import pallas as pl
from jax.experimental.pallas import tpu as pltpu
from jax.experimental.pallas import tpu_sc as plsc

B, C, H, W = 4, 96, 384, 384
Hp, Wp = H // 2, W // 2
P = B * C
NW = 32
PPW = P // NW
R = 48
NCH = Hp // R
IN_CH = R * Wp
OUT_CH = 2 * R * W
TOT = PPW * NCH

_mesh = plsc.VectorSubcoreMesh(core_axis_name="c", subcore_axis_name="s")


@functools.partial(
    pl.kernel,
    mesh=_mesh,
    out_type=jax.ShapeDtypeStruct((P * H * W,), jnp.float32),
    scratch_types=[
        pltpu.VMEM((IN_CH,), jnp.float32),
        pltpu.VMEM((IN_CH,), jnp.float32),
        pltpu.VMEM((IN_CH,), jnp.int32),
        pltpu.VMEM((IN_CH,), jnp.int32),
        pltpu.VMEM((OUT_CH,), jnp.float32),
        pltpu.VMEM((OUT_CH,), jnp.float32),
        pltpu.SemaphoreType.DMA,
        pltpu.SemaphoreType.DMA,
        pltpu.SemaphoreType.DMA,
        pltpu.SemaphoreType.DMA,
    ],
    compiler_params=pltpu.CompilerParams(needs_layout_passes=False),
)
def _unpool(x_hbm, idx_hbm, out_hbm, x0, x1, i0, i1, o0, o1, si0, si1, so0, so1):
    xs, idxs, outs = [x0, x1], [i0, i1], [o0, o1]
    sis, sos = [si0, si1], [so0, so1]
    wid = lax.axis_index("s") * 2 + lax.axis_index("c")
    base_plane = wid * PPW

    def in_off(g):
        return (base_plane + (g >> 2)) * (Hp * Wp) + (g & 3) * IN_CH

    def out_off(g):
        return (base_plane + (g >> 2)) * (H * W) + (g & 3) * OUT_CH

    def issue_in(g, b):
        off = in_off(g)
        pltpu.async_copy(x_hbm.at[pl.ds(off, IN_CH)], xs[b], sis[b])
        pltpu.async_copy(idx_hbm.at[pl.ds(off, IN_CH)], idxs[b], sis[b])

    def wait_in(g, b):
        off = in_off(g)
        pltpu.make_async_copy(x_hbm.at[pl.ds(off, IN_CH)], xs[b], sis[b]).wait()
        pltpu.make_async_copy(idx_hbm.at[pl.ds(off, IN_CH)], idxs[b], sis[b]).wait()

    def wait_out(g, b):
        pltpu.make_async_copy(outs[b], out_hbm.at[pl.ds(out_off(g), OUT_CH)], sos[b]).wait()

    def chunk(g, b, first=False, issue_next=True):
        if issue_next:
            issue_in(g + 1, 1 - b)
        if not first:
            wait_out(g, b)
        out_v = outs[b]
        zeros = jnp.zeros((16,), jnp.float32)

        def zbody(k, _):
            out_v[pl.ds(k * 16, 16)] = zeros
            return ()

        lax.fori_loop(0, OUT_CH // 16, zbody, (), unroll=8)
        wait_in(g, b)
        x_v, idx_v = xs[b], idxs[b]
        base = jnp.full((16,), (g & 3) * OUT_CH, jnp.int32)

        def sbody(k, _):
            iv = idx_v[pl.ds(k * 16, 16)] - base
            xv = x_v[pl.ds(k * 16, 16)]
            plsc.store_scatter(out_v, [iv], xv)
            return ()

        lax.fori_loop(0, IN_CH // 16, sbody, (), unroll=8)
        pltpu.async_copy(out_v, out_hbm.at[pl.ds(out_off(g), OUT_CH)], sos[b])

    issue_in(0, 0)
    chunk(0, 0, first=True)
    chunk(1, 1, first=True)

    def pair(g2, _):
        g = g2 * 2
        chunk(g, 0)
        chunk(g + 1, 1)
        return ()

    lax.fori_loop(1, TOT // 2 - 1, pair, ())

    chunk(TOT - 2, 0)
    chunk(TOT - 1, 1, issue_next=False)

    wait_out(TOT - 2, 0)
    wait_out(TOT - 1, 1)


def kernel(x, indices):
    out = _unpool(x.reshape(-1), indices.reshape(-1))
    return out.reshape(B, C, H, W)


# speedup vs baseline: 153.7780x; 1.5291x over previous
import functools

import jax
import jax.numpy as jnp
from jax import lax
from jax.experimental import pallas as pl
from jax.experimental.pallas import tpu as pltpu
from jax.experimental.pallas import tpu_sc as plsc

B, C, H, W = 4, 96, 384, 384
Hp, Wp = H // 2, W // 2
P = B * C
NW = 32
PPW = P // NW
R = 48
NCH = Hp // R
TOT = PPW * NCH
VPR = Wp // 16

_mesh = plsc.VectorSubcoreMesh(core_axis_name="c", subcore_axis_name="s")


@functools.partial(
    pl.kernel,
    mesh=_mesh,
    out_type=jax.ShapeDtypeStruct((P, H, W), jnp.float32),
    scratch_types=[
        pltpu.VMEM((R, Wp), jnp.float32),
        pltpu.VMEM((R, Wp), jnp.float32),
        pltpu.VMEM((R, Wp), jnp.int32),
        pltpu.VMEM((R, Wp), jnp.int32),
        pltpu.VMEM((2 * R, W), jnp.float32),
        pltpu.VMEM((2 * R, W), jnp.float32),
        pltpu.SemaphoreType.DMA,
        pltpu.SemaphoreType.DMA,
        pltpu.SemaphoreType.DMA,
        pltpu.SemaphoreType.DMA,
    ],
    compiler_params=pltpu.CompilerParams(needs_layout_passes=False),
)
def _unpool(x_hbm, idx_hbm, out_hbm, x0, x1, i0, i1, o0, o1, si0, si1, so0, so1):
    xs, idxs, outs = [x0, x1], [i0, i1], [o0, o1]
    sis, sos = [si0, si1], [so0, so1]
    wid = lax.axis_index("s") * 2 + lax.axis_index("c")
    base_plane = wid * PPW

    def refs_of(g):
        plane = base_plane + (g >> 2)
        r0 = (g & 3) * R
        return plane, r0

    def issue_in(g, b):
        plane, r0 = refs_of(g)
        pltpu.async_copy(x_hbm.at[plane, pl.ds(r0, R), :], xs[b], sis[b])
        pltpu.async_copy(idx_hbm.at[plane, pl.ds(r0, R), :], idxs[b], sis[b])

    def wait_in(g, b):
        plane, r0 = refs_of(g)
        pltpu.make_async_copy(x_hbm.at[plane, pl.ds(r0, R), :], xs[b], sis[b]).wait()
        pltpu.make_async_copy(idx_hbm.at[plane, pl.ds(r0, R), :], idxs[b], sis[b]).wait()

    def out_ref_of(g):
        plane, r0 = refs_of(g)
        return out_hbm.at[plane, pl.ds(2 * r0, 2 * R), :]

    def wait_out(g, b):
        pltpu.make_async_copy(outs[b], out_ref_of(g), sos[b]).wait()

    def chunk(g, b, first=False, issue_next=True):
        if issue_next:
            issue_in(g + 1, 1 - b)
        if not first:
            wait_out(g, b)
        out_v = outs[b]
        zeros = jnp.zeros((16,), jnp.float32)

        def zbody(h, _):
            for cz in range(W // 16):
                out_v[h, pl.ds(cz * 16, 16)] = zeros
            return ()

        lax.fori_loop(0, 2 * R, zbody, ())
        wait_in(g, b)
        x_v, idx_v = xs[b], idxs[b]
        _, r0 = refs_of(g)

        def sbody(il, _):
            rbase = (r0 + il) * (2 * W)
            hbase = jnp.full((16,), 2 * il, jnp.int32)
            for cv in range(VPR):
                iv = idx_v[il, pl.ds(cv * 16, 16)]
                xv = x_v[il, pl.ds(cv * 16, 16)]
                rel = iv - rbase
                q = rel >> 7
                dr = (q + 1) >> 2
                wv = rel - ((dr << 8) + (dr << 7))
                hv = hbase + dr
                plsc.store_scatter(out_v, [hv, wv], xv)
            return ()

        lax.fori_loop(0, R, sbody, ())
        pltpu.async_copy(out_v, out_ref_of(g), sos[b])

    issue_in(0, 0)
    chunk(0, 0, first=True)
    chunk(1, 1, first=True)

    def pair(g2, _):
        g = g2 * 2
        chunk(g, 0)
        chunk(g + 1, 1)
        return ()

    lax.fori_loop(1, TOT // 2 - 1, pair, ())

    chunk(TOT - 2, 0)
    chunk(TOT - 1, 1, issue_next=False)

    wait_out(TOT - 2, 0)
    wait_out(TOT - 1, 1)


def kernel(x, indices):
    out = _unpool(x.reshape(P, Hp, Wp), indices.reshape(P, Hp, Wp))
    return out.reshape(B, C, H, W)


# speedup vs baseline: 338.5082x; 2.2013x over previous
import functools

import jax
import jax.numpy as jnp
from jax import lax
from jax.experimental import pallas as pl
from jax.experimental.pallas import tpu as pltpu
from jax.experimental.pallas import tpu_sc as plsc

B, C, H, W = 4, 96, 384, 384
Hp, Wp = H // 2, W // 2
P = B * C
NW = 32
PPW = P // NW
R = 48
NCH = Hp // R
TOT = PPW * NCH
VPR = Wp // 16

_mesh = plsc.VectorSubcoreMesh(core_axis_name="c", subcore_axis_name="s")


@functools.partial(
    pl.kernel,
    mesh=_mesh,
    out_type=jax.ShapeDtypeStruct((P, H, W), jnp.float32),
    scratch_types=[
        pltpu.VMEM((R, Wp), jnp.float32),
        pltpu.VMEM((R, Wp), jnp.float32),
        pltpu.VMEM((R, Wp), jnp.int32),
        pltpu.VMEM((R, Wp), jnp.int32),
        pltpu.VMEM((2 * R, W), jnp.float32),
        pltpu.VMEM((2 * R, W), jnp.float32),
        pltpu.SemaphoreType.DMA,
        pltpu.SemaphoreType.DMA,
        pltpu.SemaphoreType.DMA,
        pltpu.SemaphoreType.DMA,
    ],
    compiler_params=pltpu.CompilerParams(needs_layout_passes=False),
)
def _unpool(x_hbm, idx_hbm, out_hbm, x0, x1, i0, i1, o0, o1, si0, si1, so0, so1):
    xs, idxs, outs = [x0, x1], [i0, i1], [o0, o1]
    sis, sos = [si0, si1], [so0, so1]
    wid = lax.axis_index("s") * 2 + lax.axis_index("c")
    base_plane = wid * PPW

    def refs_of(g):
        plane = base_plane + (g >> 2)
        r0 = (g & 3) * R
        return plane, r0

    def issue_in(g, b):
        plane, r0 = refs_of(g)
        pltpu.async_copy(x_hbm.at[plane, pl.ds(r0, R), :], xs[b], sis[b])
        pltpu.async_copy(idx_hbm.at[plane, pl.ds(r0, R), :], idxs[b], sis[b])

    def wait_in(g, b):
        plane, r0 = refs_of(g)
        pltpu.make_async_copy(x_hbm.at[plane, pl.ds(r0, R), :], xs[b], sis[b]).wait()
        pltpu.make_async_copy(idx_hbm.at[plane, pl.ds(r0, R), :], idxs[b], sis[b]).wait()

    def out_ref_of(g):
        plane, r0 = refs_of(g)
        return out_hbm.at[plane, pl.ds(2 * r0, 2 * R), :]

    def wait_out(g, b):
        pltpu.make_async_copy(outs[b], out_ref_of(g), sos[b]).wait()

    def chunk(g, b, first=False, issue_next=True):
        if issue_next:
            issue_in(g + 1, 1 - b)
        if not first:
            wait_out(g, b)
        out_v = outs[b]
        zeros = jnp.zeros((16,), jnp.float32)

        @plsc.parallel_loop(0, 2 * R, unroll=2)
        def _zero(h):
            for cz in range(W // 16):
                out_v[h, pl.ds(cz * 16, 16)] = zeros

        wait_in(g, b)
        x_v, idx_v = xs[b], idxs[b]
        _, r0 = refs_of(g)

        @plsc.parallel_loop(0, R, unroll=2)
        def _scatter(il):
            rbase = (r0 + il) * (2 * W)
            hbase = jnp.full((16,), 2 * il, jnp.int32)
            for cv in range(VPR):
                iv = idx_v[il, pl.ds(cv * 16, 16)]
                xv = x_v[il, pl.ds(cv * 16, 16)]
                rel = iv - rbase
                q = rel >> 7
                dr = (q + 1) >> 2
                wv = rel - ((dr << 8) + (dr << 7))
                hv = hbase + dr
                plsc.store_scatter(out_v, [hv, wv], xv)
        pltpu.async_copy(out_v, out_ref_of(g), sos[b])

    issue_in(0, 0)
    chunk(0, 0, first=True)
    chunk(1, 1, first=True)

    def pair(g2, _):
        g = g2 * 2
        chunk(g, 0)
        chunk(g + 1, 1)
        return ()

    lax.fori_loop(1, TOT // 2 - 1, pair, ())

    chunk(TOT - 2, 0)
    chunk(TOT - 1, 1, issue_next=False)

    wait_out(TOT - 2, 0)
    wait_out(TOT - 1, 1)


def kernel(x, indices):
    out = _unpool(x.reshape(P, Hp, Wp), indices.reshape(P, Hp, Wp))
    return out.reshape(B, C, H, W)
